# baseline (speedup 1.0000x reference)
import functools

import jax
import jax.numpy as jnp
from jax import lax
from jax.experimental import pallas as pl
from jax.experimental.pallas import tpu as pltpu
from jax.experimental.pallas import tpu_sc as plsc

_VOCAB = 1000
_EMBED = 128
_CONTEXT = 4
_BATCH = 16384
_HIDDEN = 128

_NC, _NS = 2, 16
_NW = _NC * _NS
_ROWS_PER_W = _BATCH // _NW


def _sc_gather(emb, idx):
    mesh = plsc.VectorSubcoreMesh(core_axis_name="c", subcore_axis_name="s")

    @functools.partial(
        pl.kernel,
        out_type=jax.ShapeDtypeStruct((_BATCH, _EMBED), jnp.float32),
        mesh=mesh,
        scratch_types=[
            pltpu.VMEM((_ROWS_PER_W,), jnp.int32),
            pltpu.VMEM((_ROWS_PER_W, _EMBED), jnp.float32),
            pltpu.SemaphoreType.DMA,
        ],
    )
    def body(emb_hbm, idx_hbm, out_hbm, idx_v, rows_v, sem):
        wid = lax.axis_index("s") * _NC + lax.axis_index("c")
        base = wid * _ROWS_PER_W
        pltpu.sync_copy(idx_hbm.at[pl.ds(base, _ROWS_PER_W)], idx_v)
        pltpu.async_copy(emb_hbm.at[idx_v], rows_v, sem).wait()
        pltpu.sync_copy(rows_v, out_hbm.at[pl.ds(base, _ROWS_PER_W)])

    return body(emb, idx)


_BB = 2048


def _mlp_body(e_ref, w1_ref, b1_ref, w2t_ref, b2_ref, out_ref, ht_ref):
    c = pl.program_id(1)

    @pl.when(c == 0)
    def _():
        ht = lax.dot_general(w1_ref[...], e_ref[...], (((0,), (1,)), ((), ())),
                             preferred_element_type=jnp.float32)
        ht_ref[...] = jnp.maximum(ht + b1_ref[...], 0.0).astype(jnp.bfloat16)

    out_ref[0] = (
        jnp.dot(w2t_ref[0], ht_ref[...], preferred_element_type=jnp.float32)
        + b2_ref[0]
    )


def _tc_mlp(e, w1, b1, w2t, b2):
    grid = (_BATCH // _BB, _CONTEXT)
    return pl.pallas_call(
        _mlp_body,
        grid=grid,
        in_specs=[
            pl.BlockSpec((_BB, _EMBED), lambda i, c: (i, 0)),
            pl.BlockSpec((_EMBED, _HIDDEN), lambda i, c: (0, 0)),
            pl.BlockSpec((_HIDDEN, 1), lambda i, c: (0, 0)),
            pl.BlockSpec((1, _VOCAB, _HIDDEN), lambda i, c: (c, 0, 0)),
            pl.BlockSpec((1, _VOCAB, 1), lambda i, c: (c, 0, 0)),
        ],
        out_specs=pl.BlockSpec((1, _VOCAB, _BB), lambda i, c: (c, 0, i)),
        out_shape=jax.ShapeDtypeStruct((_CONTEXT, _VOCAB, _BATCH), jnp.float32),
        scratch_shapes=[pltpu.VMEM((_HIDDEN, _BB), jnp.bfloat16)],
    )(e, w1, b1, w2t, b2)


def kernel(x, emb, W1, b1, W2, b2):
    e = _sc_gather(emb, x.astype(jnp.int32))
    w2t = W2.T.reshape(_CONTEXT, _VOCAB, _HIDDEN).astype(jnp.bfloat16)
    out = _tc_mlp(e, W1, b1.reshape(_HIDDEN, 1), w2t,
                  b2.reshape(_CONTEXT, _VOCAB, 1))
    return out.transpose(2, 0, 1)


# speedup vs baseline: 1.0993x; 1.0993x over previous
import functools

import jax
import jax.numpy as jnp
from jax import lax
from jax.experimental import pallas as pl
from jax.experimental.pallas import tpu as pltpu
from jax.experimental.pallas import tpu_sc as plsc

_VOCAB = 1000
_EMBED = 128
_CONTEXT = 4
_BATCH = 16384
_HIDDEN = 128

_NC, _NS = 2, 16
_NW = _NC * _NS
_ROWS_PER_W = _BATCH // _NW


def _sc_gather(emb, idx):
    mesh = plsc.VectorSubcoreMesh(core_axis_name="c", subcore_axis_name="s")

    @functools.partial(
        pl.kernel,
        out_type=jax.ShapeDtypeStruct((_BATCH, _EMBED), jnp.float32),
        mesh=mesh,
        scratch_types=[
            pltpu.VMEM((_ROWS_PER_W,), jnp.int32),
            pltpu.VMEM((_ROWS_PER_W, _EMBED), jnp.float32),
            pltpu.SemaphoreType.DMA,
        ],
    )
    def body(emb_hbm, idx_hbm, out_hbm, idx_v, rows_v, sem):
        wid = lax.axis_index("s") * _NC + lax.axis_index("c")
        base = wid * _ROWS_PER_W
        pltpu.sync_copy(idx_hbm.at[pl.ds(base, _ROWS_PER_W)], idx_v)
        pltpu.async_copy(emb_hbm.at[idx_v], rows_v, sem).wait()
        pltpu.sync_copy(rows_v, out_hbm.at[pl.ds(base, _ROWS_PER_W)])

    return body(emb, idx)


_BB = 1024


def _mlp_body(e_ref, w1_ref, b1_ref, w2t_ref, b2_ref, out_ref):
    ht = lax.dot_general(w1_ref[...], e_ref[...], (((0,), (1,)), ((), ())),
                         preferred_element_type=jnp.float32)
    ht = jnp.maximum(ht + b1_ref[...], 0.0).astype(jnp.bfloat16)
    for c in range(_CONTEXT):
        out_ref[c] = (
            jnp.dot(w2t_ref[c], ht, preferred_element_type=jnp.float32)
            + b2_ref[c]
        )


def _tc_mlp(e, w1, b1, w2t, b2):
    grid = (_BATCH // _BB,)
    return pl.pallas_call(
        _mlp_body,
        grid=grid,
        in_specs=[
            pl.BlockSpec((_BB, _EMBED), lambda i: (i, 0)),
            pl.BlockSpec((_EMBED, _HIDDEN), lambda i: (0, 0)),
            pl.BlockSpec((_HIDDEN, 1), lambda i: (0, 0)),
            pl.BlockSpec((_CONTEXT, _VOCAB, _HIDDEN), lambda i: (0, 0, 0)),
            pl.BlockSpec((_CONTEXT, _VOCAB, 1), lambda i: (0, 0, 0)),
        ],
        out_specs=pl.BlockSpec((_CONTEXT, _VOCAB, _BB), lambda i: (0, 0, i)),
        out_shape=jax.ShapeDtypeStruct((_CONTEXT, _VOCAB, _BATCH), jnp.float32),
        compiler_params=pltpu.CompilerParams(
            dimension_semantics=("parallel",)),
    )(e, w1, b1, w2t, b2)


def kernel(x, emb, W1, b1, W2, b2):
    e = _sc_gather(emb, x.astype(jnp.int32))
    w2t = W2.T.reshape(_CONTEXT, _VOCAB, _HIDDEN).astype(jnp.bfloat16)
    out = _tc_mlp(e, W1, b1.reshape(_HIDDEN, 1), w2t,
                  b2.reshape(_CONTEXT, _VOCAB, 1))
    return out.transpose(2, 0, 1)
